# baseline (speedup 1.0000x reference)
import functools

import jax
import jax.numpy as jnp
from jax import lax
from jax.experimental import pallas as pl
from jax.experimental.pallas import tpu as pltpu
from jax.experimental.pallas import tpu_sc as plsc

_B, _P, _C, _O = 32, 24564, 21, 32
_PADP = 24576
_G, _L = 192, 128
_CR = 16
_NCH = _G // _CR
_CR1 = 32
_NCH1 = _G // _CR1
_THRESH = 0.5
_NEGPOS = 3
_VAR0, _VAR1 = 0.1, 0.2
_BIG = 2**30
_NCHUNK = _PADP // 16
_UNROLL = 8
_NSTEP = _NCHUNK // _UNROLL


def _match_kernel(tgt_ref, pri_ref, bto_ref, bti_ref, bpi_ref):
    f32 = jnp.float32
    t_x1 = [tgt_ref[0, t, 0] for t in range(_O)]
    t_y1 = [tgt_ref[0, t, 1] for t in range(_O)]
    t_x2 = [tgt_ref[0, t, 2] for t in range(_O)]
    t_y2 = [tgt_ref[0, t, 3] for t in range(_O)]
    tarea = [(t_x2[t] - t_x1[t]) * (t_y2[t] - t_y1[t]) for t in range(_O)]

    vmax_t = [jnp.full((1, _L), -2.0, f32) for _ in range(_O)]
    vlin_t = [jnp.full((1, _L), _BIG, jnp.int32) for _ in range(_O)]

    for c in range(_NCH1):
        r0 = c * _CR1
        pcx = pri_ref[0, r0:r0 + _CR1, :]
        pcy = pri_ref[1, r0:r0 + _CR1, :]
        pw = pri_ref[2, r0:r0 + _CR1, :]
        ph = pri_ref[3, r0:r0 + _CR1, :]
        px1 = pcx - pw * 0.5
        py1 = pcy - ph * 0.5
        px2 = pcx + pw * 0.5
        py2 = pcy + ph * 0.5
        parea = pw * ph
        lin = ((lax.broadcasted_iota(jnp.int32, (_CR1, _L), 0) + r0) * _L
               + lax.broadcasted_iota(jnp.int32, (_CR1, _L), 1))
        valid = lin < _P

        ngrp = 4
        per = _O // ngrp
        bt_os = [jnp.full((_CR1, _L), -2.0, f32) for _ in range(ngrp)]
        bt_is = [jnp.zeros((_CR1, _L), jnp.int32) for _ in range(ngrp)]
        for t in range(_O):
            g = t // per
            iw = jnp.maximum(jnp.minimum(t_x2[t], px2) - jnp.maximum(t_x1[t], px1), 0.0)
            ih = jnp.maximum(jnp.minimum(t_y2[t], py2) - jnp.maximum(t_y1[t], py1), 0.0)
            inter = iw * ih
            ovl = inter / (tarea[t] + parea - inter)
            ovl = jnp.where(valid, ovl, -1.0)
            take = ovl > bt_os[g]
            bt_is[g] = jnp.where(take, t, bt_is[g])
            bt_os[g] = jnp.where(take, ovl, bt_os[g])
            colmax = jnp.max(ovl, axis=0, keepdims=True)
            colidx = jnp.min(jnp.where(ovl == colmax, lin, _BIG),
                             axis=0, keepdims=True)
            upd = colmax > vmax_t[t]
            vlin_t[t] = jnp.where(upd, colidx, vlin_t[t])
            vmax_t[t] = jnp.where(upd, colmax, vmax_t[t])

        bt_o, bt_i = bt_os[0], bt_is[0]
        for g in range(1, ngrp):
            take = bt_os[g] > bt_o
            bt_i = jnp.where(take, bt_is[g], bt_i)
            bt_o = jnp.where(take, bt_os[g], bt_o)

        bto_ref[0, r0:r0 + _CR1, :] = bt_o
        bti_ref[0, r0:r0 + _CR1, :] = bt_i

    lane = lax.broadcasted_iota(jnp.int32, (1, _L), 1)
    row = jnp.zeros((1, _L), jnp.int32)
    for t in range(_O):
        mx = jnp.max(vmax_t[t])
        mi = jnp.min(jnp.where(vmax_t[t] == mx, vlin_t[t], _BIG))
        row = jnp.where(lane == t, mi, row)
    bpi_ref[0] = row


def _loss_kernel(tgt_ref, bpi_ref, loc_ref, conf_ref, pri_ref, bto_ref,
                 bti_ref, out_ref, lc_ref):
    f32 = jnp.float32
    t_x1 = [tgt_ref[0, t, 0] for t in range(_O)]
    t_y1 = [tgt_ref[0, t, 1] for t in range(_O)]
    t_x2 = [tgt_ref[0, t, 2] for t in range(_O)]
    t_y2 = [tgt_ref[0, t, 3] for t in range(_O)]
    t_lab = [tgt_ref[0, t, 4] for t in range(_O)]
    bpi = [bpi_ref[0, 0, t] for t in range(_O)]

    acc_l = jnp.zeros((_CR, _L), f32)
    acc_pce = jnp.zeros((_CR, _L), f32)
    acc_np = jnp.zeros((_CR, _L), f32)

    def sl1(d):
        a = jnp.abs(d)
        return jnp.where(a < 1.0, 0.5 * d * d, a - 0.5)

    for c in range(_NCH):
        r0 = c * _CR
        lin = ((lax.broadcasted_iota(jnp.int32, (_CR, _L), 0) + r0) * _L
               + lax.broadcasted_iota(jnp.int32, (_CR, _L), 1))
        valid = lin < _P
        bt_o = bto_ref[0, r0:r0 + _CR, :]
        bt_i = bti_ref[0, r0:r0 + _CR, :]

        forced = jnp.full((_CR, _L), -1, jnp.int32)
        for t in range(_O):
            forced = jnp.where(lin == bpi[t], t, forced)
        isf = forced >= 0
        bt_i = jnp.where(isf, forced, bt_i)
        bt_o = jnp.where(isf, 2.0, bt_o)

        mx1 = jnp.zeros((_CR, _L), f32)
        my1 = jnp.zeros((_CR, _L), f32)
        mx2 = jnp.zeros((_CR, _L), f32)
        my2 = jnp.zeros((_CR, _L), f32)
        mlab = jnp.zeros((_CR, _L), f32)
        for t in range(_O):
            s = bt_i == t
            mx1 = jnp.where(s, t_x1[t], mx1)
            my1 = jnp.where(s, t_y1[t], my1)
            mx2 = jnp.where(s, t_x2[t], mx2)
            my2 = jnp.where(s, t_y2[t], my2)
            mlab = jnp.where(s, t_lab[t], mlab)

        pos = jnp.logical_and(bt_o >= _THRESH, mlab > 0.0)
        pos = jnp.logical_and(pos, valid)

        pcx = pri_ref[0, r0:r0 + _CR, :]
        pcy = pri_ref[1, r0:r0 + _CR, :]
        pw = pri_ref[2, r0:r0 + _CR, :]
        ph = pri_ref[3, r0:r0 + _CR, :]
        g_cx = ((mx1 + mx2) * 0.5 - pcx) / (_VAR0 * pw)
        g_cy = ((my1 + my2) * 0.5 - pcy) / (_VAR0 * ph)
        safe_w = jnp.where(pos, (mx2 - mx1) / pw, 1.0)
        safe_h = jnp.where(pos, (my2 - my1) / ph, 1.0)
        g_w = jnp.log(safe_w) / _VAR1
        g_h = jnp.log(safe_h) / _VAR1

        l_elem = (sl1(loc_ref[0, 0, r0:r0 + _CR, :] - g_cx)
                  + sl1(loc_ref[0, 1, r0:r0 + _CR, :] - g_cy)
                  + sl1(loc_ref[0, 2, r0:r0 + _CR, :] - g_w)
                  + sl1(loc_ref[0, 3, r0:r0 + _CR, :] - g_h))
        acc_l = acc_l + jnp.where(pos, l_elem, 0.0)

        rowmax = conf_ref[0, 0, r0:r0 + _CR, :]
        for ci in range(1, _C):
            rowmax = jnp.maximum(rowmax, conf_ref[0, ci, r0:r0 + _CR, :])
        conf_t = jnp.where(bt_o < _THRESH, 0, mlab.astype(jnp.int32))
        sexp = jnp.zeros((_CR, _L), f32)
        gath = jnp.zeros((_CR, _L), f32)
        for ci in range(_C):
            x = conf_ref[0, ci, r0:r0 + _CR, :]
            sexp = sexp + jnp.exp(x - rowmax)
            gath = jnp.where(conf_t == ci, x, gath)
        ce = jnp.log(sexp) + rowmax - gath
        ce = jnp.where(valid, ce, 0.0)

        acc_pce = acc_pce + jnp.where(pos, ce, 0.0)
        acc_np = acc_np + jnp.where(pos, 1.0, 0.0)

        loss_c = jnp.maximum(jnp.where(pos, 0.0, ce), 0.0)
        lc_ref[0, r0:r0 + _CR, :] = loss_c

    loss_l = jnp.sum(acc_l)
    pos_ce = jnp.sum(acc_pce)
    npos = jnp.sum(acc_np)

    lane = lax.broadcasted_iota(jnp.int32, (1, _L), 1)
    row = jnp.where(lane == 0, loss_l,
                    jnp.where(lane == 1, pos_ce,
                              jnp.where(lane == 2, npos, 0.0)))
    out_ref[0] = row


def _hsum(vec):
    tot = vec[0]
    for q in range(1, 16):
        tot = tot + vec[q]
    return tot


def _sc_select(loss_hbm, part_hbm, out_hbm, vals, kv, ov):
    w = lax.axis_index("s") * 2 + lax.axis_index("c")
    pltpu.sync_copy(loss_hbm.at[w], vals)
    pltpu.sync_copy(part_hbm.at[w, 0], kv)
    npos_f = kv[pl.ds(0, 16)][2]
    k_sc = jnp.minimum(_NEGPOS * npos_f.astype(jnp.int32), _P - 1)

    zero_i = jnp.zeros((16,), jnp.int32)

    def outer(i, t_acc):
        cand = t_acc | lax.shift_left(jnp.int32(1), 30 - i)

        def inner(j, accs):
            a = list(accs)
            for u in range(_UNROLL):
                v = vals[pl.ds(j * (16 * _UNROLL) + u * 16, 16)]
                b = lax.bitcast_convert_type(v, jnp.int32)
                a[u % 4] = a[u % 4] + jnp.where(b >= cand, 1, 0)
            return tuple(a)

        a0, a1, a2, a3 = lax.fori_loop(0, _NSTEP, inner,
                                       (zero_i, zero_i, zero_i, zero_i))
        cnt = _hsum(a0 + a1 + a2 + a3)
        return jnp.where(cnt >= k_sc, cand, t_acc)

    tbits = lax.fori_loop(0, 31, outer, jnp.int32(0))

    zero_f = jnp.zeros((16,), jnp.float32)

    def fin(j, carry):
        s = list(carry[:4])
        m = list(carry[4:])
        for u in range(_UNROLL):
            v = vals[pl.ds(j * (16 * _UNROLL) + u * 16, 16)]
            b = lax.bitcast_convert_type(v, jnp.int32)
            gt = b > tbits
            s[u % 4] = s[u % 4] + jnp.where(gt, v, 0.0)
            m[u % 4] = m[u % 4] + jnp.where(gt, 1, 0)
        return tuple(s) + tuple(m)

    fr = lax.fori_loop(0, _NSTEP, fin,
                       (zero_f, zero_f, zero_f, zero_f,
                        zero_i, zero_i, zero_i, zero_i))
    s_tot = _hsum(fr[0] + fr[1] + fr[2] + fr[3])
    m_tot = _hsum(fr[4] + fr[5] + fr[6] + fr[7])
    t_vec = lax.bitcast_convert_type(zero_i + tbits, jnp.float32)
    r = k_sc - m_tot
    tie = jnp.where((zero_i + r) > 0, t_vec * r.astype(jnp.float32), 0.0)

    lane = lax.broadcasted_iota(jnp.int32, (16,), 0)
    ov[...] = jnp.where(lane == 0, s_tot + tie, 0.0)
    pltpu.sync_copy(ov, out_hbm.at[w])


def kernel(loc_data, conf_data, priors, targets):
    pad = _PADP - _P
    dummy = jnp.tile(jnp.array([[-10.0, -10.0, 0.1, 0.1]], jnp.float32), (pad, 1))
    pri = jnp.concatenate([priors, dummy], axis=0).T.reshape(4, _G, _L)
    loc = jnp.pad(loc_data, ((0, 0), (0, pad), (0, 0))).transpose(0, 2, 1)
    loc = loc.reshape(_B, 4, _G, _L)
    conf = jnp.pad(conf_data, ((0, 0), (0, pad), (0, 0))).transpose(0, 2, 1)
    conf = conf.reshape(_B, _C, _G, _L)

    bto, bti, bpi = pl.pallas_call(
        _match_kernel,
        grid=(_B,),
        in_specs=[
            pl.BlockSpec((1, _O, 5), lambda b: (b, 0, 0),
                         memory_space=pltpu.SMEM),
            pl.BlockSpec((4, _G, _L), lambda b: (0, 0, 0)),
        ],
        out_specs=[
            pl.BlockSpec((1, _G, _L), lambda b: (b, 0, 0)),
            pl.BlockSpec((1, _G, _L), lambda b: (b, 0, 0)),
            pl.BlockSpec((1, 1, _L), lambda b: (b, 0, 0)),
        ],
        out_shape=[
            jax.ShapeDtypeStruct((_B, _G, _L), jnp.float32),
            jax.ShapeDtypeStruct((_B, _G, _L), jnp.int32),
            jax.ShapeDtypeStruct((_B, 1, _L), jnp.int32),
        ],
    )(targets, pri)

    partial, loss_c = pl.pallas_call(
        _loss_kernel,
        grid=(_B,),
        in_specs=[
            pl.BlockSpec((1, _O, 5), lambda b: (b, 0, 0),
                         memory_space=pltpu.SMEM),
            pl.BlockSpec((1, 1, _L), lambda b: (b, 0, 0),
                         memory_space=pltpu.SMEM),
            pl.BlockSpec((1, 4, _G, _L), lambda b: (b, 0, 0, 0)),
            pl.BlockSpec((1, _C, _G, _L), lambda b: (b, 0, 0, 0)),
            pl.BlockSpec((4, _G, _L), lambda b: (0, 0, 0)),
            pl.BlockSpec((1, _G, _L), lambda b: (b, 0, 0)),
            pl.BlockSpec((1, _G, _L), lambda b: (b, 0, 0)),
        ],
        out_specs=[
            pl.BlockSpec((1, 1, _L), lambda b: (b, 0, 0)),
            pl.BlockSpec((1, _G, _L), lambda b: (b, 0, 0)),
        ],
        out_shape=[
            jax.ShapeDtypeStruct((_B, 1, _L), jnp.float32),
            jax.ShapeDtypeStruct((_B, _G, _L), jnp.float32),
        ],
    )(targets, bpi, loc, conf, pri, bto, bti)

    part = partial.reshape(_B, _L)
    loss_l = jnp.sum(part[:, 0])
    pos_ce = jnp.sum(part[:, 1])
    npos = part[:, 2]

    mesh = plsc.VectorSubcoreMesh(core_axis_name="c", subcore_axis_name="s")
    sc_fn = functools.partial(
        pl.kernel,
        out_type=jax.ShapeDtypeStruct((_B, 16), jnp.float32),
        mesh=mesh,
        scratch_types=[
            pltpu.VMEM((_PADP,), jnp.float32),
            pltpu.VMEM((_L,), jnp.float32),
            pltpu.VMEM((16,), jnp.float32),
        ],
    )(_sc_select)
    negrow = sc_fn(loss_c.reshape(_B, _PADP), partial)

    loss_c_sum = pos_ce + jnp.sum(negrow[:, 0])
    n = jnp.maximum(jnp.sum(npos), 1.0)
    return (loss_l / n, loss_c_sum / n)


# speedup vs baseline: 1.1023x; 1.1023x over previous
import functools

import jax
import jax.numpy as jnp
from jax import lax
from jax.experimental import pallas as pl
from jax.experimental.pallas import tpu as pltpu
from jax.experimental.pallas import tpu_sc as plsc

_B, _P, _C, _O = 32, 24564, 21, 32
_PADP = 24576
_G, _L = 192, 128
_THRESH = 0.5
_NEGPOS = 3
_VAR0, _VAR1 = 0.1, 0.2
_BIG = 2**30
_NCHUNK = _PADP // 16


def _image_kernel(tgt_ref, loc_ref, conf_ref, pri_ref, out_ref, lc_ref):
    f32 = jnp.float32
    lin = (lax.broadcasted_iota(jnp.int32, (_G, _L), 0) * _L
           + lax.broadcasted_iota(jnp.int32, (_G, _L), 1))
    valid = lin < _P

    pcx = pri_ref[0]
    pcy = pri_ref[1]
    pw = pri_ref[2]
    ph = pri_ref[3]
    px1 = pcx - pw * 0.5
    py1 = pcy - ph * 0.5
    px2 = pcx + pw * 0.5
    py2 = pcy + ph * 0.5
    parea = pw * ph

    t_x1 = [tgt_ref[0, t, 0] for t in range(_O)]
    t_y1 = [tgt_ref[0, t, 1] for t in range(_O)]
    t_x2 = [tgt_ref[0, t, 2] for t in range(_O)]
    t_y2 = [tgt_ref[0, t, 3] for t in range(_O)]
    t_lab = [tgt_ref[0, t, 4] for t in range(_O)]

    bt_ovl = jnp.full((_G, _L), -2.0, f32)
    bt_idx = jnp.zeros((_G, _L), jnp.int32)
    bpi = []
    for t in range(_O):
        iw = jnp.maximum(jnp.minimum(t_x2[t], px2) - jnp.maximum(t_x1[t], px1), 0.0)
        ih = jnp.maximum(jnp.minimum(t_y2[t], py2) - jnp.maximum(t_y1[t], py1), 0.0)
        inter = iw * ih
        tarea = (t_x2[t] - t_x1[t]) * (t_y2[t] - t_y1[t])
        ovl = inter / (tarea + parea - inter)
        ovl = jnp.where(valid, ovl, -1.0)
        take = ovl > bt_ovl
        bt_idx = jnp.where(take, t, bt_idx)
        bt_ovl = jnp.where(take, ovl, bt_ovl)
        mx = jnp.max(ovl)
        bpi.append(jnp.min(jnp.where(ovl == mx, lin, _BIG)))

    for t in range(_O):
        m = lin == bpi[t]
        bt_idx = jnp.where(m, t, bt_idx)
        bt_ovl = jnp.where(m, 2.0, bt_ovl)

    mx1 = jnp.zeros((_G, _L), f32)
    my1 = jnp.zeros((_G, _L), f32)
    mx2 = jnp.zeros((_G, _L), f32)
    my2 = jnp.zeros((_G, _L), f32)
    mlab = jnp.zeros((_G, _L), f32)
    for t in range(_O):
        s = bt_idx == t
        mx1 = jnp.where(s, t_x1[t], mx1)
        my1 = jnp.where(s, t_y1[t], my1)
        mx2 = jnp.where(s, t_x2[t], mx2)
        my2 = jnp.where(s, t_y2[t], my2)
        mlab = jnp.where(s, t_lab[t], mlab)

    pos = jnp.logical_and(bt_ovl >= _THRESH, mlab > 0.0)
    pos = jnp.logical_and(pos, valid)

    g_cx = ((mx1 + mx2) * 0.5 - pcx) / (_VAR0 * pw)
    g_cy = ((my1 + my2) * 0.5 - pcy) / (_VAR0 * ph)
    safe_w = jnp.where(pos, (mx2 - mx1) / pw, 1.0)
    safe_h = jnp.where(pos, (my2 - my1) / ph, 1.0)
    g_w = jnp.log(safe_w) / _VAR1
    g_h = jnp.log(safe_h) / _VAR1

    def sl1(d):
        a = jnp.abs(d)
        return jnp.where(a < 1.0, 0.5 * d * d, a - 0.5)

    l_elem = (sl1(loc_ref[0, 0] - g_cx) + sl1(loc_ref[0, 1] - g_cy)
              + sl1(loc_ref[0, 2] - g_w) + sl1(loc_ref[0, 3] - g_h))
    loss_l = jnp.sum(jnp.where(pos, l_elem, 0.0))

    rowmax = conf_ref[0, 0]
    for c in range(1, _C):
        rowmax = jnp.maximum(rowmax, conf_ref[0, c])
    conf_t = jnp.where(bt_ovl < _THRESH, 0, mlab.astype(jnp.int32))
    sexp = jnp.zeros((_G, _L), f32)
    gath = jnp.zeros((_G, _L), f32)
    for c in range(_C):
        x = conf_ref[0, c]
        sexp = sexp + jnp.exp(x - rowmax)
        gath = jnp.where(conf_t == c, x, gath)
    ce = jnp.log(sexp) + rowmax - gath
    ce = jnp.where(valid, ce, 0.0)

    pos_ce = jnp.sum(jnp.where(pos, ce, 0.0))
    npos = jnp.sum(pos.astype(jnp.int32))

    loss_c = jnp.maximum(jnp.where(jnp.logical_or(pos, jnp.logical_not(valid)),
                                   0.0, ce), 0.0)
    lc_ref[0] = loss_c

    lane = lax.broadcasted_iota(jnp.int32, (1, _L), 1)
    row = jnp.where(lane == 0, loss_l,
                    jnp.where(lane == 1, pos_ce,
                              jnp.where(lane == 2, npos.astype(f32), 0.0)))
    out_ref[0] = row


def _hsum(vec):
    tot = vec[0]
    for q in range(1, 16):
        tot = tot + vec[q]
    return tot


_UNROLL = 8
_NSTEP = _NCHUNK // _UNROLL


def _sc_select(loss_hbm, part_hbm, out_hbm, vals, kv, ov):
    w = lax.axis_index("s") * 2 + lax.axis_index("c")
    pltpu.sync_copy(loss_hbm.at[w], vals)
    pltpu.sync_copy(part_hbm.at[w, 0], kv)
    npos_f = kv[pl.ds(0, 16)][2]
    k_sc = jnp.minimum(_NEGPOS * npos_f.astype(jnp.int32), _P - 1)

    zero_i = jnp.zeros((16,), jnp.int32)

    def outer(i, t_acc):
        cand = t_acc | lax.shift_left(jnp.int32(1), 30 - i)

        def inner(j, accs):
            a = list(accs)
            for u in range(_UNROLL):
                v = vals[pl.ds(j * (16 * _UNROLL) + u * 16, 16)]
                b = lax.bitcast_convert_type(v, jnp.int32)
                a[u % 4] = a[u % 4] + jnp.where(b >= cand, 1, 0)
            return tuple(a)

        a0, a1, a2, a3 = lax.fori_loop(0, _NSTEP, inner,
                                       (zero_i, zero_i, zero_i, zero_i))
        cnt = _hsum(a0 + a1 + a2 + a3)
        return jnp.where(cnt >= k_sc, cand, t_acc)

    tbits = lax.fori_loop(0, 31, outer, jnp.int32(0))

    zero_f = jnp.zeros((16,), jnp.float32)

    def fin(j, carry):
        s = list(carry[:4])
        m = list(carry[4:])
        for u in range(_UNROLL):
            v = vals[pl.ds(j * (16 * _UNROLL) + u * 16, 16)]
            b = lax.bitcast_convert_type(v, jnp.int32)
            gt = b > tbits
            s[u % 4] = s[u % 4] + jnp.where(gt, v, 0.0)
            m[u % 4] = m[u % 4] + jnp.where(gt, 1, 0)
        return tuple(s) + tuple(m)

    fr = lax.fori_loop(0, _NSTEP, fin,
                       (zero_f, zero_f, zero_f, zero_f,
                        zero_i, zero_i, zero_i, zero_i))
    s_tot = _hsum(fr[0] + fr[1] + fr[2] + fr[3])
    m_tot = _hsum(fr[4] + fr[5] + fr[6] + fr[7])
    t_vec = lax.bitcast_convert_type(zero_i + tbits, jnp.float32)
    r = k_sc - m_tot
    tie = jnp.where((zero_i + r) > 0, t_vec * r.astype(jnp.float32), 0.0)

    lane = lax.broadcasted_iota(jnp.int32, (16,), 0)
    ov[...] = jnp.where(lane == 0, s_tot + tie, 0.0)
    pltpu.sync_copy(ov, out_hbm.at[w])


def kernel(loc_data, conf_data, priors, targets):
    pad = _PADP - _P
    dummy = jnp.tile(jnp.array([[-10.0, -10.0, 0.1, 0.1]], jnp.float32), (pad, 1))
    pri = jnp.concatenate([priors, dummy], axis=0).T.reshape(4, _G, _L)
    loc = jnp.pad(loc_data, ((0, 0), (0, pad), (0, 0))).transpose(0, 2, 1)
    loc = loc.reshape(_B, 4, _G, _L)
    conf = jnp.pad(conf_data, ((0, 0), (0, pad), (0, 0))).transpose(0, 2, 1)
    conf = conf.reshape(_B, _C, _G, _L)

    partial, loss_c = pl.pallas_call(
        _image_kernel,
        grid=(_B,),
        in_specs=[
            pl.BlockSpec((1, _O, 5), lambda b: (b, 0, 0),
                         memory_space=pltpu.SMEM),
            pl.BlockSpec((1, 4, _G, _L), lambda b: (b, 0, 0, 0)),
            pl.BlockSpec((1, _C, _G, _L), lambda b: (b, 0, 0, 0)),
            pl.BlockSpec((4, _G, _L), lambda b: (0, 0, 0)),
        ],
        out_specs=[
            pl.BlockSpec((1, 1, _L), lambda b: (b, 0, 0)),
            pl.BlockSpec((1, _G, _L), lambda b: (b, 0, 0)),
        ],
        out_shape=[
            jax.ShapeDtypeStruct((_B, 1, _L), jnp.float32),
            jax.ShapeDtypeStruct((_B, _G, _L), jnp.float32),
        ],
    )(targets, loc, conf, pri)

    part = partial.reshape(_B, _L)
    loss_l = jnp.sum(part[:, 0])
    pos_ce = jnp.sum(part[:, 1])
    npos = part[:, 2]

    mesh = plsc.VectorSubcoreMesh(core_axis_name="c", subcore_axis_name="s")
    sc_fn = functools.partial(
        pl.kernel,
        out_type=jax.ShapeDtypeStruct((_B, 16), jnp.float32),
        mesh=mesh,
        scratch_types=[
            pltpu.VMEM((_PADP,), jnp.float32),
            pltpu.VMEM((_L,), jnp.float32),
            pltpu.VMEM((16,), jnp.float32),
        ],
    )(_sc_select)
    negrow = sc_fn(loss_c.reshape(_B, _PADP), partial)

    loss_c_sum = pos_ce + jnp.sum(negrow[:, 0])
    n = jnp.maximum(jnp.sum(npos), 1.0)
    return (loss_l / n, loss_c_sum / n)
